# Initial kernel scaffold; baseline (speedup 1.0000x reference)
import functools

import jax
import jax.numpy as jnp
from jax import lax
from jax.experimental import pallas as pl
from jax.experimental.pallas import tpu as pltpu
from jax.experimental.pallas import tpu_sc as plsc

GAMMA = 12.0
HIDDEN = 128
BATCH = 16384
L = 16
NCHUNK = HIDDEN // L

_INFO = plsc.get_sparse_core_info()
NC = _INFO.num_cores
NS = _INFO.num_subcores
NW = NC * NS
BPW = BATCH // NW
C = 128
NITER = BPW // C


def _rsqrt16(x):
    i = lax.bitcast_convert_type(x, jnp.int32)
    i = jnp.int32(0x5F3759DF) - lax.shift_right_arithmetic(i, jnp.int32(1))
    y = lax.bitcast_convert_type(i, jnp.float32)
    for _ in range(3):
        y = y * (1.5 - 0.5 * x * y * y)
    return y


def _make_sc_kernel():
    mesh = plsc.VectorSubcoreMesh(core_axis_name="c", subcore_axis_name="s")

    @functools.partial(
        pl.kernel,
        mesh=mesh,
        out_type=jax.ShapeDtypeStruct((BATCH,), jnp.float32),
        scratch_types=[
            pltpu.VMEM((C,), jnp.int32),
            pltpu.VMEM((C,), jnp.int32),
            pltpu.VMEM((C,), jnp.int32),
            pltpu.VMEM((C, HIDDEN), jnp.float32),
            pltpu.VMEM((C, HIDDEN), jnp.float32),
            pltpu.VMEM((C, HIDDEN), jnp.float32),
            pltpu.VMEM((BPW,), jnp.float32),
            pltpu.SemaphoreType.DMA,
        ],
    )
    def score_kernel(hidx_hbm, ridx_hbm, tidx_hbm, ent_hbm, rel_hbm, out_hbm,
                     hidx_v, ridx_v, tidx_v, hrows_v, rrows_v, trows_v,
                     out_v, sem):
        wid = lax.axis_index("s") * NC + lax.axis_index("c")
        wbase = wid * BPW

        for chunk in range(NITER):
            base = wbase + chunk * C
            pltpu.sync_copy(hidx_hbm.at[pl.ds(base, C)], hidx_v)
            pltpu.sync_copy(ridx_hbm.at[pl.ds(base, C)], ridx_v)
            pltpu.sync_copy(tidx_hbm.at[pl.ds(base, C)], tidx_v)
            cp_h = pltpu.async_copy(ent_hbm.at[hidx_v], hrows_v, sem)
            cp_r = pltpu.async_copy(rel_hbm.at[ridx_v], rrows_v, sem)
            cp_t = pltpu.async_copy(ent_hbm.at[tidx_v], trows_v, sem)
            cp_h.wait()
            cp_r.wait()
            cp_t.wait()

            obase = chunk * C

            def triple(t, _):
                h = [hrows_v[t, pl.ds(L * j, L)] for j in range(NCHUNK)]
                tt = [trows_v[t, pl.ds(L * j, L)] for j in range(NCHUNK)]
                rr = [rrows_v[t, pl.ds(L * j, L)] for j in range(NCHUNK)]
                h2 = h[0] * h[0]
                t2 = tt[0] * tt[0]
                for j in range(1, NCHUNK):
                    h2 = h2 + h[j] * h[j]
                    t2 = t2 + tt[j] * tt[j]
                h2s = jnp.sum(h2)
                t2s = jnp.sum(t2)
                inh = _rsqrt16(lax.broadcast_in_dim(h2s, (L,), ()))
                int_ = _rsqrt16(lax.broadcast_in_dim(t2s, (L,), ()))
                acc = jnp.abs(h[0] * inh + rr[0] - tt[0] * int_)
                for j in range(1, NCHUNK):
                    acc = acc + jnp.abs(h[j] * inh + rr[j] - tt[j] * int_)
                out_v[obase + t] = GAMMA - jnp.sum(acc)
                return 0

            lax.fori_loop(0, C, triple, 0)

        pltpu.sync_copy(out_v, out_hbm.at[pl.ds(wbase, BPW)])

    return score_kernel


_SC_KERNEL = _make_sc_kernel()


@jax.jit
def kernel(sample, entity_embedding, relation_embedding):
    hidx = sample[:, 0].astype(jnp.int32)
    ridx = sample[:, 1].astype(jnp.int32)
    tidx = sample[:, 2].astype(jnp.int32)
    score = _SC_KERNEL(hidx, ridx, tidx, entity_embedding, relation_embedding)
    return score[:, None]


# speedup vs baseline: 1.0669x; 1.0669x over previous
import functools

import jax
import jax.numpy as jnp
from jax import lax
from jax.experimental import pallas as pl
from jax.experimental.pallas import tpu as pltpu
from jax.experimental.pallas import tpu_sc as plsc

GAMMA = 12.0
HIDDEN = 128
BATCH = 16384
L = 16
NCHUNK = HIDDEN // L

_INFO = plsc.get_sparse_core_info()
NC = _INFO.num_cores
NS = _INFO.num_subcores
NW = NC * NS
BPW = BATCH // NW
C = 128
NITER = BPW // C


def _allsum16(x, lane):
    del lane
    return lax.broadcast_in_dim(jnp.sum(x), (L,), ())


def _rsqrt16(x):
    i = lax.bitcast_convert_type(x, jnp.int32)
    i = jnp.int32(0x5F3759DF) - lax.shift_right_arithmetic(i, jnp.int32(1))
    y = lax.bitcast_convert_type(i, jnp.float32)
    for _ in range(3):
        y = y * (1.5 - 0.5 * x * y * y)
    return y


def _make_sc_kernel():
    mesh = plsc.VectorSubcoreMesh(core_axis_name="c", subcore_axis_name="s")

    @functools.partial(
        pl.kernel,
        mesh=mesh,
        compiler_params=pltpu.CompilerParams(needs_layout_passes=False),
        out_type=jax.ShapeDtypeStruct((BATCH,), jnp.float32),
        scratch_types=[
            pltpu.VMEM((C,), jnp.int32),
            pltpu.VMEM((C,), jnp.int32),
            pltpu.VMEM((C,), jnp.int32),
            pltpu.VMEM((C, HIDDEN), jnp.float32),
            pltpu.VMEM((C, HIDDEN), jnp.float32),
            pltpu.VMEM((C, HIDDEN), jnp.float32),
            pltpu.VMEM((BPW,), jnp.float32),
            pltpu.SemaphoreType.DMA,
        ],
    )
    def score_kernel(hidx_hbm, ridx_hbm, tidx_hbm, ent_hbm, rel_hbm, out_hbm,
                     hidx_v, ridx_v, tidx_v, hrows_v, rrows_v, trows_v,
                     out_v, sem):
        wid = lax.axis_index("s") * NC + lax.axis_index("c")
        wbase = wid * BPW

        for chunk in range(NITER):
            base = wbase + chunk * C
            pltpu.sync_copy(hidx_hbm.at[pl.ds(base, C)], hidx_v)
            pltpu.sync_copy(ridx_hbm.at[pl.ds(base, C)], ridx_v)
            pltpu.sync_copy(tidx_hbm.at[pl.ds(base, C)], tidx_v)
            cp_h = pltpu.async_copy(ent_hbm.at[hidx_v], hrows_v, sem)
            cp_r = pltpu.async_copy(rel_hbm.at[ridx_v], rrows_v, sem)
            cp_t = pltpu.async_copy(ent_hbm.at[tidx_v], trows_v, sem)
            cp_h.wait()
            cp_r.wait()
            cp_t.wait()

            obase = chunk * C
            lane = lax.iota(jnp.int32, L)

            def triple(t, sv):
                h = [hrows_v[t, pl.ds(L * j, L)] for j in range(NCHUNK)]
                tt = [trows_v[t, pl.ds(L * j, L)] for j in range(NCHUNK)]
                rr = [rrows_v[t, pl.ds(L * j, L)] for j in range(NCHUNK)]
                h2 = h[0] * h[0]
                t2 = tt[0] * tt[0]
                for j in range(1, NCHUNK):
                    h2 = h2 + h[j] * h[j]
                    t2 = t2 + tt[j] * tt[j]
                inh = _rsqrt16(_allsum16(h2, lane))
                int_ = _rsqrt16(_allsum16(t2, lane))
                acc = jnp.abs(h[0] * inh + rr[0] - tt[0] * int_)
                for j in range(1, NCHUNK):
                    acc = acc + jnp.abs(h[j] * inh + rr[j] - tt[j] * int_)
                s = GAMMA - _allsum16(acc, lane)
                sv = jnp.where(lane == t % L, s, sv)

                @pl.when(t % L == L - 1)
                def _():
                    out_v[pl.ds(obase + t - (L - 1), L)] = sv

                return sv

            lax.fori_loop(0, C, triple, jnp.zeros((L,), jnp.float32))

        pltpu.sync_copy(out_v, out_hbm.at[pl.ds(wbase, BPW)])

    return score_kernel


_SC_KERNEL = _make_sc_kernel()


@jax.jit
def kernel(sample, entity_embedding, relation_embedding):
    hidx = sample[:, 0].astype(jnp.int32)
    ridx = sample[:, 1].astype(jnp.int32)
    tidx = sample[:, 2].astype(jnp.int32)
    score = _SC_KERNEL(hidx, ridx, tidx, entity_embedding, relation_embedding)
    return score[:, None]


# speedup vs baseline: 1.6485x; 1.5451x over previous
import functools

import jax
import jax.numpy as jnp
from jax import lax
from jax.experimental import pallas as pl
from jax.experimental.pallas import tpu as pltpu
from jax.experimental.pallas import tpu_sc as plsc

GAMMA = 12.0
HIDDEN = 128
BATCH = 16384
L = 16
NCHUNK = HIDDEN // L

_INFO = plsc.get_sparse_core_info()
NC = _INFO.num_cores
NS = _INFO.num_subcores
NW = NC * NS
BPW = BATCH // NW
C = 128
NITER = BPW // C


def _rsqrt16(x):
    i = lax.bitcast_convert_type(x, jnp.int32)
    i = jnp.int32(0x5F3759DF) - lax.shift_right_arithmetic(i, jnp.int32(1))
    y = lax.bitcast_convert_type(i, jnp.float32)
    for _ in range(3):
        y = y * (1.5 - 0.5 * x * y * y)
    return y


def _make_sc_kernel():
    mesh = plsc.VectorSubcoreMesh(core_axis_name="c", subcore_axis_name="s")

    @functools.partial(
        pl.kernel,
        mesh=mesh,
        compiler_params=pltpu.CompilerParams(needs_layout_passes=False),
        out_type=jax.ShapeDtypeStruct((BATCH,), jnp.float32),
        scratch_types=[
            pltpu.VMEM((NITER, 3, C), jnp.int32),
            pltpu.VMEM((2, C, HIDDEN), jnp.float32),
            pltpu.VMEM((2, C, HIDDEN), jnp.float32),
            pltpu.VMEM((2, C, HIDDEN), jnp.float32),
            pltpu.VMEM((BPW,), jnp.float32),
            pltpu.SemaphoreType.DMA,
            pltpu.SemaphoreType.DMA,
        ],
    )
    def score_kernel(idx_hbm, ent_hbm, rel_hbm, out_hbm,
                     idx_v, hrows_v, rrows_v, trows_v, out_v, sem0, sem1):
        wid = lax.axis_index("s") * NC + lax.axis_index("c")
        wbase = wid * BPW
        sems = (sem0, sem1)

        pltpu.sync_copy(idx_hbm.at[wid], idx_v)

        def start(c):
            buf = c % 2
            sem = sems[buf]
            return (
                pltpu.async_copy(ent_hbm.at[idx_v.at[c, 0]], hrows_v.at[buf], sem),
                pltpu.async_copy(rel_hbm.at[idx_v.at[c, 1]], rrows_v.at[buf], sem),
                pltpu.async_copy(ent_hbm.at[idx_v.at[c, 2]], trows_v.at[buf], sem),
            )

        pending = start(0)
        for chunk in range(NITER):
            cur = pending
            if chunk + 1 < NITER:
                pending = start(chunk + 1)
            for cp in cur:
                cp.wait()

            buf = chunk % 2
            obase = chunk * C
            lane = lax.iota(jnp.int32, L)

            def triple(t, sv):
                h = [hrows_v[buf, t, pl.ds(L * j, L)] for j in range(NCHUNK)]
                tt = [trows_v[buf, t, pl.ds(L * j, L)] for j in range(NCHUNK)]
                rr = [rrows_v[buf, t, pl.ds(L * j, L)] for j in range(NCHUNK)]
                h2 = h[0] * h[0]
                t2 = tt[0] * tt[0]
                for j in range(1, NCHUNK):
                    h2 = h2 + h[j] * h[j]
                    t2 = t2 + tt[j] * tt[j]
                inh = _rsqrt16(lax.broadcast_in_dim(jnp.sum(h2), (L,), ()))
                int_ = _rsqrt16(lax.broadcast_in_dim(jnp.sum(t2), (L,), ()))
                acc = jnp.abs(h[0] * inh + rr[0] - tt[0] * int_)
                for j in range(1, NCHUNK):
                    acc = acc + jnp.abs(h[j] * inh + rr[j] - tt[j] * int_)
                s = GAMMA - lax.broadcast_in_dim(jnp.sum(acc), (L,), ())
                sv = jnp.where(lane == t % L, s, sv)

                @pl.when(t % L == L - 1)
                def _():
                    out_v[pl.ds(obase + t - (L - 1), L)] = sv

                return sv

            lax.fori_loop(0, C, triple, jnp.zeros((L,), jnp.float32), unroll=4)

        pltpu.sync_copy(out_v, out_hbm.at[pl.ds(wbase, BPW)])

    return score_kernel


_SC_KERNEL = _make_sc_kernel()


@jax.jit
def kernel(sample, entity_embedding, relation_embedding):
    s32 = sample.astype(jnp.int32)
    idx = jnp.stack(
        [s32[:, 0].reshape(NW, NITER, C),
         s32[:, 1].reshape(NW, NITER, C),
         s32[:, 2].reshape(NW, NITER, C)], axis=2)
    score = _SC_KERNEL(idx, entity_embedding, relation_embedding)
    return score[:, None]
